# baseline (speedup 1.0000x reference)
import functools

import jax
import jax.numpy as jnp
from jax import lax
from jax.experimental import pallas as pl
from jax.experimental.pallas import tpu as pltpu
from jax.experimental.pallas import tpu_sc as plsc

_NUM_POINTS = 1000000
_PT_DIM = 64
_N_REP = 16384
_NC, _NS = 2, 16
_NW = _NC * _NS
_ROWS_PER_W = _PT_DIM // _NW
_CHUNK = 128
_NCHUNK = _N_REP // _CHUNK
_GROUP = 8


@functools.cache
def _make_scatter():
    mesh = plsc.VectorSubcoreMesh(
        core_axis_name="c", subcore_axis_name="s", num_cores=_NC, num_subcores=_NS
    )

    @functools.partial(
        pl.kernel,
        out_type=jax.ShapeDtypeStruct((_PT_DIM, _NUM_POINTS), jnp.float32),
        mesh=mesh,
        compiler_params=pltpu.CompilerParams(use_tc_tiling_on_sc=False),
        scratch_types=[
            pltpu.VMEM((_NCHUNK, _CHUNK), jnp.int32),
            pltpu.VMEM((_ROWS_PER_W, _N_REP), jnp.float32),
            pltpu.SemaphoreType.DMA,
            pltpu.SemaphoreType.DMA,
        ],
    )
    def _scatter_cols(ptsT_hbm, rvT_hbm, idx_hbm, outT_hbm, idx_v, val_v, row_sem, sc_sem):
        w = lax.axis_index("s") * _NC + lax.axis_index("c")
        base = w * _ROWS_PER_W
        row_copies = [
            pltpu.async_copy(ptsT_hbm.at[base + j], outT_hbm.at[base + j], row_sem)
            for j in range(_ROWS_PER_W)
        ]
        pltpu.sync_copy(idx_hbm, idx_v)
        pltpu.sync_copy(rvT_hbm.at[pl.ds(base, _ROWS_PER_W)], val_v)
        for c in row_copies:
            c.wait()
        for j in range(_ROWS_PER_W):
            row = outT_hbm.at[base + j]

            @pl.loop(0, _NCHUNK // _GROUP)
            def _(g):
                for b in range(_GROUP):
                    q = g * _GROUP + b
                    pltpu.async_copy(
                        val_v.at[j, pl.ds(q * _CHUNK, _CHUNK)],
                        row.at[idx_v.at[q]],
                        sc_sem,
                    )
                for b in range(_GROUP):
                    pltpu.make_async_copy(
                        ptsT_hbm.at[0, pl.ds(0, _CHUNK)],
                        val_v.at[j, pl.ds(0, _CHUNK)],
                        sc_sem,
                    ).wait()

    return _scatter_cols


def kernel(pts, rand_vals, idx):
    ptsT = pts.T
    rvT = rand_vals.T
    idx2 = idx.astype(jnp.int32).reshape(_NCHUNK, _CHUNK)
    outT = _make_scatter()(ptsT, rvT, idx2)
    return outT.T


# speedup vs baseline: 1.5122x; 1.5122x over previous
import functools

import jax
import jax.numpy as jnp
from jax import lax
from jax.experimental import pallas as pl
from jax.experimental.pallas import tpu as pltpu
from jax.experimental.pallas import tpu_sc as plsc

_NUM_POINTS = 1000000
_PT_DIM = 64
_N_REP = 16384
_NC, _NS = 2, 16
_NW = _NC * _NS
_ROWS_PER_W = _PT_DIM // _NW
_CHUNK = 128
_NCHUNK = _N_REP // _CHUNK
_GROUP = 8
_C = 10000
_NCOPY = _NUM_POINTS // _C
_NBUF = 4


@functools.cache
def _make_scatter():
    mesh = plsc.VectorSubcoreMesh(
        core_axis_name="c", subcore_axis_name="s", num_cores=_NC, num_subcores=_NS
    )

    @functools.partial(
        pl.kernel,
        out_type=jax.ShapeDtypeStruct((_PT_DIM, _NUM_POINTS), jnp.float32),
        mesh=mesh,
        compiler_params=pltpu.CompilerParams(use_tc_tiling_on_sc=False),
        scratch_types=[
            pltpu.VMEM((_NCHUNK, _CHUNK), jnp.int32),
            pltpu.VMEM((_ROWS_PER_W, _N_REP), jnp.float32),
            pltpu.VMEM((_NBUF, _C), jnp.float32),
            pltpu.SemaphoreType.DMA,
            pltpu.SemaphoreType.DMA,
            pltpu.SemaphoreType.DMA,
        ],
    )
    def _scatter_cols(
        ptsT_hbm, rvT_hbm, idx_hbm, outT_hbm, idx_v, val_v, buf, ld_sem, st_sem, sc_sem
    ):
        w = lax.axis_index("s") * _NC + lax.axis_index("c")
        base = w * _ROWS_PER_W
        pltpu.sync_copy(idx_hbm, idx_v)
        pltpu.sync_copy(rvT_hbm.at[pl.ds(base, _ROWS_PER_W)], val_v)

        for j in range(_ROWS_PER_W):
            r = base + j

            @pl.loop(0, _NCOPY // _NBUF)
            def _(g):
                for b in range(_NBUF):
                    off = (g * _NBUF + b) * _C
                    pltpu.async_copy(
                        ptsT_hbm.at[r, pl.ds(off, _C)], buf.at[b], ld_sem
                    )
                for b in range(_NBUF):
                    pltpu.make_async_copy(
                        ptsT_hbm.at[r, pl.ds(0, _C)], buf.at[b], ld_sem
                    ).wait()
                for b in range(_NBUF):
                    off = (g * _NBUF + b) * _C
                    pltpu.async_copy(
                        buf.at[b], outT_hbm.at[r, pl.ds(off, _C)], st_sem
                    )
                for b in range(_NBUF):
                    pltpu.make_async_copy(
                        buf.at[b], outT_hbm.at[r, pl.ds(0, _C)], st_sem
                    ).wait()

            @pl.loop(0, _NCHUNK // _GROUP)
            def _(g):
                for b in range(_GROUP):
                    q = g * _GROUP + b
                    pltpu.async_copy(
                        val_v.at[j, pl.ds(q * _CHUNK, _CHUNK)],
                        outT_hbm.at[r].at[idx_v.at[q]],
                        sc_sem,
                    )

        for j in range(_ROWS_PER_W):
            pltpu.make_async_copy(
                ptsT_hbm.at[0, pl.ds(0, _N_REP)], val_v.at[j], sc_sem
            ).wait()

    return _scatter_cols


def kernel(pts, rand_vals, idx):
    ptsT = pts.T
    rvT = rand_vals.T
    idx2 = idx.astype(jnp.int32).reshape(_NCHUNK, _CHUNK)
    outT = _make_scatter()(ptsT, rvT, idx2)
    return outT.T
